# baseline (speedup 1.0000x reference)
import jax
import jax.numpy as jnp
from jax.experimental import pallas as pl
from jax.experimental.pallas import tpu as pltpu

_B, _N = 16, 1024
_NODE_DIM, _COND_DIM, _EDGE_DIM, _K = 256, 512, 256, 32


def _dot_bf16(x, y, dn):
    return jax.lax.dot_general(x.astype(jnp.bfloat16), y.astype(jnp.bfloat16),
                               dn, preferred_element_type=jnp.float32)


def _topk32_desc(x):
    slabs = [x[32 * a:32 * (a + 1), :] for a in range(32)]
    k = 2
    while k <= 32:
        j = k // 2
        while j >= 1:
            for i in range(32):
                if (i & j) == 0:
                    p = i | j
                    hi = jnp.maximum(slabs[i], slabs[p])
                    lo = jnp.minimum(slabs[i], slabs[p])
                    if (i & k) == 0:
                        slabs[i], slabs[p] = hi, lo
                    else:
                        slabs[i], slabs[p] = lo, hi
            j //= 2
        k *= 2
    c = 32
    while c > 1:
        half = c // 2
        slabs = [jnp.maximum(slabs[a][:half], slabs[31 - a][half:])
                 for a in range(32)]
        j = 16
        while j >= 1:
            for i in range(32):
                if (i & j) == 0:
                    p = i | j
                    hi = jnp.maximum(slabs[i], slabs[p])
                    lo = jnp.minimum(slabs[i], slabs[p])
                    slabs[i], slabs[p] = hi, lo
            j //= 2
        c = half
    return jnp.concatenate(slabs, axis=0)


def _fused_body(nf_ref, cond_ref, va_ref, vb_ref, g1_ref, b1_ref,
                v2_ref, g2_ref, b2_ref, out_ref, lt_ref):
    s = pl.program_id(0)

    @pl.when(s > 0)
    def _consume():
        prev = jax.lax.rem(s + 1, 2)
        lt = lt_ref[prev]
        m = jnp.max(lt, axis=0)
        ssum = jnp.sum(jnp.exp(lt - m[None, :]), axis=0)
        v = _topk32_desc(lt)
        w = jnp.exp(v - m[None, :]) / ssum[None, :]
        out_ref[...] = w.T

    @pl.when(s < _B)
    def _produce():
        nf = nf_ref[0]
        cond = cond_ref[0]
        va = va_ref[...]
        vb = vb_ref[...]
        v2 = v2_ref[...]
        g1 = g1_ref[0]
        g2 = g2_ref[0]
        n1 = jnp.sqrt(jnp.sum(va * va, axis=1) + jnp.sum(vb * vb, axis=1))
        wa = g1[:, None] * va / n1[:, None]
        wb = g1[:, None] * vb / n1[:, None]
        n2 = jnp.sqrt(jnp.sum(v2 * v2, axis=1))
        w2 = g2[:, None] * v2 / n2[:, None]
        dn = (((1,), (1,)), ((), ()))
        x = jnp.concatenate(
            [nf, jnp.broadcast_to(cond, (nf.shape[0], cond.shape[1]))], axis=1)
        w1 = jnp.concatenate([wa, wb], axis=1)
        h = jnp.maximum(_dot_bf16(x, w1, dn) + b1_ref[...], 0.0)
        jf = jnp.maximum(_dot_bf16(h, w2, dn) + b2_ref[...], 0.0)
        lt_ref[jax.lax.rem(s, 2)] = _dot_bf16(jf, jf, dn)


@jax.jit
def kernel(node_feats, cond_feats, v1, g1, b1, v2, g2, b2):
    va = v1[:, :_NODE_DIM]
    vb = v1[:, _NODE_DIM:]
    g1r, b1r = g1.reshape(1, -1), b1.reshape(1, -1)
    g2r, b2r = g2.reshape(1, -1), b2.reshape(1, -1)

    cst = lambda s: (0, 0)
    out = pl.pallas_call(
        _fused_body,
        grid=(_B + 1,),
        in_specs=[
            pl.BlockSpec((1, _N, _NODE_DIM),
                         lambda s: (jax.lax.min(s, _B - 1), 0, 0)),
            pl.BlockSpec((1, 1, _COND_DIM),
                         lambda s: (jax.lax.min(s, _B - 1), 0, 0)),
            pl.BlockSpec((_EDGE_DIM, _NODE_DIM), cst),
            pl.BlockSpec((_EDGE_DIM, _COND_DIM), cst),
            pl.BlockSpec((1, _EDGE_DIM), cst),
            pl.BlockSpec((1, _EDGE_DIM), cst),
            pl.BlockSpec((_EDGE_DIM, _EDGE_DIM), cst),
            pl.BlockSpec((1, _EDGE_DIM), cst),
            pl.BlockSpec((1, _EDGE_DIM), cst),
        ],
        out_specs=pl.BlockSpec((_N, _K), lambda s: (jax.lax.max(s - 1, 0), 0)),
        out_shape=jax.ShapeDtypeStruct((_B * _N, _K), jnp.float32),
        scratch_shapes=[pltpu.VMEM((2, _N, _N), jnp.float32)],
    )(node_feats, cond_feats.reshape(_B, 1, _COND_DIM), va, vb, g1r, b1r,
      v2, g2r, b2r)
    return out


# speedup vs baseline: 1.4876x; 1.4876x over previous
import jax
import jax.numpy as jnp
from jax.experimental import pallas as pl
from jax.experimental.pallas import tpu as pltpu

_B, _N = 16, 1024
_NODE_DIM, _COND_DIM, _EDGE_DIM, _K = 256, 512, 256, 32


def _dot_bf16(x, y, dn):
    return jax.lax.dot_general(x.astype(jnp.bfloat16), y.astype(jnp.bfloat16),
                               dn, preferred_element_type=jnp.float32)


def _topk32_desc(x):
    slabs = [x[32 * a:32 * (a + 1), :] for a in range(32)]
    k = 2
    while k <= 32:
        j = k // 2
        while j >= 1:
            for i in range(32):
                if (i & j) == 0:
                    p = i | j
                    hi = jnp.maximum(slabs[i], slabs[p])
                    lo = jnp.minimum(slabs[i], slabs[p])
                    if (i & k) == 0:
                        slabs[i], slabs[p] = hi, lo
                    else:
                        slabs[i], slabs[p] = lo, hi
            j //= 2
        k *= 2
    c = 32
    while c > 1:
        half = c // 2
        slabs = [jnp.maximum(slabs[a][:half], slabs[31 - a][half:])
                 for a in range(32)]
        j = 16
        while j >= 1:
            for i in range(32):
                if (i & j) == 0:
                    p = i | j
                    hi = jnp.maximum(slabs[i], slabs[p])
                    lo = jnp.minimum(slabs[i], slabs[p])
                    slabs[i], slabs[p] = hi, lo
            j //= 2
        c = half
    return jnp.concatenate(slabs, axis=0)


def _fused_body(nf_ref, cond_ref, va_ref, vb_ref, g1_ref, b1_ref,
                v2_ref, g2_ref, b2_ref, out_ref):
    nf = nf_ref[0]
    cond = cond_ref[0]
    va = va_ref[...]
    vb = vb_ref[...]
    v2 = v2_ref[...]
    g1 = g1_ref[0]
    g2 = g2_ref[0]
    n1 = jnp.sqrt(jnp.sum(va * va, axis=1) + jnp.sum(vb * vb, axis=1))
    wa = g1[:, None] * va / n1[:, None]
    wb = g1[:, None] * vb / n1[:, None]
    n2 = jnp.sqrt(jnp.sum(v2 * v2, axis=1))
    w2 = g2[:, None] * v2 / n2[:, None]
    dn = (((1,), (1,)), ((), ()))
    h = (_dot_bf16(nf, wa, dn) + _dot_bf16(cond, wb, dn)) + b1_ref[...]
    h = jnp.maximum(h, 0.0)
    jf = jnp.maximum(_dot_bf16(h, w2, dn) + b2_ref[...], 0.0)

    lt = _dot_bf16(jf, jf, dn)
    v = _topk32_desc(lt)
    m = v[0:1, :]
    ssum = jnp.sum(jnp.exp(lt - m), axis=0)
    w = jnp.exp(v - m) / ssum[None, :]
    out_ref[...] = w.T


@jax.jit
def kernel(node_feats, cond_feats, v1, g1, b1, v2, g2, b2):
    va = v1[:, :_NODE_DIM]
    vb = v1[:, _NODE_DIM:]
    g1r, b1r = g1.reshape(1, -1), b1.reshape(1, -1)
    g2r, b2r = g2.reshape(1, -1), b2.reshape(1, -1)

    out = pl.pallas_call(
        _fused_body,
        grid=(_B,),
        in_specs=[
            pl.BlockSpec((1, _N, _NODE_DIM), lambda b: (b, 0, 0)),
            pl.BlockSpec((1, 1, _COND_DIM), lambda b: (b, 0, 0)),
            pl.BlockSpec((_EDGE_DIM, _NODE_DIM), lambda b: (0, 0)),
            pl.BlockSpec((_EDGE_DIM, _COND_DIM), lambda b: (0, 0)),
            pl.BlockSpec((1, _EDGE_DIM), lambda b: (0, 0)),
            pl.BlockSpec((1, _EDGE_DIM), lambda b: (0, 0)),
            pl.BlockSpec((_EDGE_DIM, _EDGE_DIM), lambda b: (0, 0)),
            pl.BlockSpec((1, _EDGE_DIM), lambda b: (0, 0)),
            pl.BlockSpec((1, _EDGE_DIM), lambda b: (0, 0)),
        ],
        out_specs=pl.BlockSpec((_N, _K), lambda b: (b, 0)),
        out_shape=jax.ShapeDtypeStruct((_B * _N, _K), jnp.float32),
    )(node_feats, cond_feats.reshape(_B, 1, _COND_DIM), va, vb, g1r, b1r,
      v2, g2r, b2r)
    return out


# speedup vs baseline: 1.5808x; 1.0627x over previous
import jax
import jax.numpy as jnp
from jax.experimental import pallas as pl
from jax.experimental.pallas import tpu as pltpu

_B, _N = 16, 1024
_NODE_DIM, _COND_DIM, _EDGE_DIM, _K = 256, 512, 256, 32


def _dot_bf16(x, y, dn):
    return jax.lax.dot_general(x.astype(jnp.bfloat16), y.astype(jnp.bfloat16),
                               dn, preferred_element_type=jnp.float32)


def _batcher_pairs(n):
    pairs = []
    p = 1
    while p < n:
        k = p
        while k >= 1:
            for j in range(k % p, n - k, 2 * k):
                for i in range(0, min(k, n - j - k)):
                    if (i + j) // (p * 2) == (i + j + k) // (p * 2):
                        pairs.append((i + j, i + j + k))
            k //= 2
        p *= 2
    return pairs


def _topk32_desc(x):
    slabs = [x[32 * a:32 * (a + 1), :] for a in range(32)]
    for i, p in _batcher_pairs(32):
        hi = jnp.maximum(slabs[i], slabs[p])
        lo = jnp.minimum(slabs[i], slabs[p])
        slabs[i], slabs[p] = hi, lo
    c = 32
    while c > 1:
        half = c // 2
        slabs = [jnp.maximum(slabs[a][:half], slabs[31 - a][half:])
                 for a in range(32)]
        j = 16
        while j >= 1:
            for i in range(32):
                if (i & j) == 0:
                    p = i | j
                    hi = jnp.maximum(slabs[i], slabs[p])
                    lo = jnp.minimum(slabs[i], slabs[p])
                    slabs[i], slabs[p] = hi, lo
            j //= 2
        c = half
    return jnp.concatenate(slabs, axis=0)


def _fused_body(nf_ref, cond_ref, va_ref, vb_ref, g1_ref, b1_ref,
                v2_ref, g2_ref, b2_ref, out_ref):
    nf = nf_ref[0]
    cond = cond_ref[0]
    va = va_ref[...]
    vb = vb_ref[...]
    v2 = v2_ref[...]
    g1 = g1_ref[0]
    g2 = g2_ref[0]
    n1 = jnp.sqrt(jnp.sum(va * va, axis=1) + jnp.sum(vb * vb, axis=1))
    wa = g1[:, None] * va / n1[:, None]
    wb = g1[:, None] * vb / n1[:, None]
    n2 = jnp.sqrt(jnp.sum(v2 * v2, axis=1))
    w2 = g2[:, None] * v2 / n2[:, None]
    dn = (((1,), (1,)), ((), ()))
    h = (_dot_bf16(nf, wa, dn) + _dot_bf16(cond, wb, dn)) + b1_ref[...]
    h = jnp.maximum(h, 0.0)
    jf = jnp.maximum(_dot_bf16(h, w2, dn) + b2_ref[...], 0.0)

    lt = _dot_bf16(jf, jf, dn)
    v = _topk32_desc(lt)
    m = v[0:1, :]
    ssum = jnp.sum(jnp.exp(lt - m), axis=0)
    w = jnp.exp(v - m) / ssum[None, :]
    out_ref[...] = w.T


@jax.jit
def kernel(node_feats, cond_feats, v1, g1, b1, v2, g2, b2):
    va = v1[:, :_NODE_DIM]
    vb = v1[:, _NODE_DIM:]
    g1r, b1r = g1.reshape(1, -1), b1.reshape(1, -1)
    g2r, b2r = g2.reshape(1, -1), b2.reshape(1, -1)

    out = pl.pallas_call(
        _fused_body,
        grid=(_B,),
        in_specs=[
            pl.BlockSpec((1, _N, _NODE_DIM), lambda b: (b, 0, 0)),
            pl.BlockSpec((1, 1, _COND_DIM), lambda b: (b, 0, 0)),
            pl.BlockSpec((_EDGE_DIM, _NODE_DIM), lambda b: (0, 0)),
            pl.BlockSpec((_EDGE_DIM, _COND_DIM), lambda b: (0, 0)),
            pl.BlockSpec((1, _EDGE_DIM), lambda b: (0, 0)),
            pl.BlockSpec((1, _EDGE_DIM), lambda b: (0, 0)),
            pl.BlockSpec((_EDGE_DIM, _EDGE_DIM), lambda b: (0, 0)),
            pl.BlockSpec((1, _EDGE_DIM), lambda b: (0, 0)),
            pl.BlockSpec((1, _EDGE_DIM), lambda b: (0, 0)),
        ],
        out_specs=pl.BlockSpec((_N, _K), lambda b: (b, 0)),
        out_shape=jax.ShapeDtypeStruct((_B * _N, _K), jnp.float32),
    )(node_feats, cond_feats.reshape(_B, 1, _COND_DIM), va, vb, g1r, b1r,
      v2, g2r, b2r)
    return out
